# baseline (speedup 1.0000x reference)
import functools

import jax
import jax.numpy as jnp
from jax import lax
from jax.experimental import pallas as pl
from jax.experimental.pallas import tpu as pltpu
from jax.experimental.pallas import tpu_sc as plsc

N = 10000
D = 128
E = 320000

NC = 2
NS = 16
NW = NC * NS

CHUNK = 32
NBUF = 8
GL = 4
IL = NBUF - 1
CPW = 320
EPW = CPW * CHUNK
E_PAD = EPW * NW

ROWS_PER_SUB = N // NS


@functools.partial(
    pl.kernel,
    out_type=jax.ShapeDtypeStruct((NC, NS, ROWS_PER_SUB, D), jnp.float32),
    mesh=plsc.VectorSubcoreMesh(core_axis_name="c", subcore_axis_name="s"),
    scratch_types=[
        [pltpu.VMEM((2, CHUNK), jnp.int32) for _ in range(NBUF)],
        [pltpu.VMEM((1, CHUNK), jnp.float32) for _ in range(NBUF)],
        [pltpu.VMEM((CHUNK, D), jnp.float32) for _ in range(NBUF)],
        pltpu.VMEM_SHARED((N, D), jnp.float32),
        [pltpu.SemaphoreType.DMA for _ in range(NBUF)],
        [pltpu.SemaphoreType.DMA for _ in range(NBUF)],
        [pltpu.SemaphoreType.DMA for _ in range(NBUF)],
    ],
)
def _sc_aggregate(x_hbm, ed_hbm, att_hbm, out_hbm,
                  pks, atts, bufs, ego_sh, isems, gsems, ssems):
    c = lax.axis_index("c")
    s = lax.axis_index("s")
    wid = s * NC + c

    zvec = jnp.zeros((16,), jnp.float32)

    def _zero_row(i, _):
        bufs[0][i // 8, pl.ds((i % 8) * 16, 16)] = zvec
        return 0

    lax.fori_loop(0, CHUNK * 8, _zero_row, 0)
    stripe = s * ROWS_PER_SUB
    nfull = ROWS_PER_SUB // CHUNK
    for k in range(nfull):
        pltpu.sync_copy(bufs[0],
                        ego_sh.at[pl.ds(stripe + k * CHUNK, CHUNK)])
    tail = ROWS_PER_SUB - nfull * CHUNK
    if tail:
        pltpu.sync_copy(bufs[0].at[pl.ds(0, tail)],
                        ego_sh.at[pl.ds(stripe + nfull * CHUNK, tail)])
    plsc.subcore_barrier()

    def _scale(attv, buf):
        def _group(g, _):
            a16 = attv[0, pl.ds(g * 16, 16)]
            for t in range(16):
                spl = jnp.take_along_axis(a16, jnp.full((16,), t, jnp.int32),
                                          axis=0)
                e = g * 16 + t
                for j in range(D // 16):
                    buf[e, pl.ds(j * 16, 16)] = buf[e, pl.ds(j * 16, 16)] * spl
            return 0

        lax.fori_loop(0, CHUNK // 16, _group, 0)

    for k in range(IL):
        pltpu.async_copy(ed_hbm.at[wid, k], pks[k], isems[k])
        pltpu.async_copy(att_hbm.at[wid, k], atts[k], isems[k])
    for k in range(GL):
        pltpu.make_async_copy(ed_hbm.at[wid, k], pks[k], isems[k]).wait()
        pltpu.make_async_copy(att_hbm.at[wid, k], atts[k], isems[k]).wait()
        pltpu.async_copy(x_hbm.at[pks[k].at[0]], bufs[k], gsems[k])

    def _step(i8, _):
        for b in range(NBUF):
            i = i8 * NBUF + b
            pk, buf = pks[b], bufs[b]
            pltpu.make_async_copy(x_hbm.at[pk.at[0]], buf, gsems[b]).wait()
            pb = (b + NBUF - 1) % NBUF

            @pl.when(i >= 1)
            def _():
                pltpu.make_async_copy(
                    bufs[pb], ego_sh.at[pks[pb].at[1]], ssems[pb]).wait()

            @pl.when(i + IL < CPW)
            def _():
                pltpu.async_copy(ed_hbm.at[wid, i + IL], pks[pb], isems[pb])
                pltpu.async_copy(att_hbm.at[wid, i + IL], atts[pb], isems[pb])

            gb = (b + GL) % NBUF

            @pl.when(i + GL < CPW)
            def _():
                pltpu.make_async_copy(
                    ed_hbm.at[wid, i + GL], pks[gb], isems[gb]).wait()
                pltpu.make_async_copy(
                    att_hbm.at[wid, i + GL], atts[gb], isems[gb]).wait()
                pltpu.async_copy(x_hbm.at[pks[gb].at[0]], bufs[gb], gsems[gb])

            _scale(atts[b], buf)
            pltpu.async_copy(buf, ego_sh.at[pk.at[1]], ssems[b], add=True)
        return 0

    lax.fori_loop(0, CPW // NBUF, _step, 0)
    lb = (CPW - 1) % NBUF
    pltpu.make_async_copy(bufs[lb], ego_sh.at[pks[lb].at[1]],
                          ssems[lb]).wait()
    plsc.subcore_barrier()

    pltpu.sync_copy(ego_sh.at[pl.ds(stripe, ROWS_PER_SUB)], out_hbm.at[c, s])


BLK = 1000


def _mlp_body(x_ref, p0_ref, p1_ref, w1_ref, b1_ref, w2_ref, b2_ref, o_ref):
    ego = p0_ref[...] + p1_ref[...]
    xv = x_ref[...]
    h1 = jnp.dot(xv + ego, w1_ref[...], preferred_element_type=jnp.float32) + b1_ref[...]
    h2 = jnp.dot(xv * ego, w2_ref[...], preferred_element_type=jnp.float32) + b2_ref[...]
    o_ref[...] = (jnp.where(h1 >= 0, h1, 0.01 * h1)
                  + jnp.where(h2 >= 0, h2, 0.01 * h2))


def _mlp(x, partials, W1, b1, W2, b2):
    grid = N // BLK
    return pl.pallas_call(
        _mlp_body,
        grid=(grid,),
        in_specs=[
            pl.BlockSpec((BLK, D), lambda i: (i, 0)),
            pl.BlockSpec((BLK, D), lambda i: (i, 0)),
            pl.BlockSpec((BLK, D), lambda i: (i + N // BLK, 0)),
            pl.BlockSpec((D, D), lambda i: (0, 0)),
            pl.BlockSpec((1, D), lambda i: (0, 0)),
            pl.BlockSpec((D, D), lambda i: (0, 0)),
            pl.BlockSpec((1, D), lambda i: (0, 0)),
        ],
        out_specs=pl.BlockSpec((BLK, D), lambda i: (i, 0)),
        out_shape=jax.ShapeDtypeStruct((N, D), jnp.float32),
    )(x, partials, partials, W1, b1, W2, b2)


def kernel(x, edge_index, attention, W1, b1, W2, b2):
    src = edge_index[0].astype(jnp.int32)
    dst = edge_index[1].astype(jnp.int32)
    pad = E_PAD - E
    src = jnp.concatenate([src, jnp.zeros((pad,), jnp.int32)])
    dst = jnp.concatenate([dst, jnp.zeros((pad,), jnp.int32)])
    att = jnp.concatenate([attention, jnp.zeros((pad,), jnp.float32)])
    ed = jnp.stack([src.reshape(NW, CPW, CHUNK),
                    dst.reshape(NW, CPW, CHUNK)], axis=2)
    attr = att.reshape(NW, CPW, 1, CHUNK)
    partials = _sc_aggregate(x, ed, attr).reshape(NC * N, D)
    return _mlp(x, partials, W1, b1.reshape(1, D), W2, b2.reshape(1, D))


# speedup vs baseline: 1.2663x; 1.2663x over previous
import functools

import jax
import jax.numpy as jnp
import numpy as np
from jax import lax
from jax.experimental import pallas as pl
from jax.experimental.pallas import tpu as pltpu
from jax.experimental.pallas import tpu_sc as plsc

N = 10000
D = 128
E = 320000

NC = 2
NS = 16
NW = NC * NS

CHUNK = 16
DB = 2
QB = 4
CPW = 640
EPW = CPW * CHUNK
E_PAD = EPW * NW

ROWS_PER_SUB = N // NS
HSTRIPE = (N // 2) // NS

_PERM = np.empty((D,), np.int32)
for _m in range(D // 32):
    _b = 32 * _m
    for _k in range(16):
        _PERM[_b + 2 * _k] = _b + _k
        _PERM[_b + 2 * _k + 1] = _b + 16 + _k


@functools.partial(
    pl.kernel,
    out_type=jax.ShapeDtypeStruct((NC, NS, ROWS_PER_SUB, D), jnp.float32),
    mesh=plsc.VectorSubcoreMesh(core_axis_name="c", subcore_axis_name="s"),
    compiler_params=pltpu.CompilerParams(needs_layout_passes=False),
    scratch_types=[
        [pltpu.VMEM((CHUNK,), jnp.int32) for _ in range(QB)],
        [pltpu.VMEM((CHUNK,), jnp.int32) for _ in range(QB)],
        [pltpu.VMEM((CHUNK,), jnp.float32) for _ in range(QB)],
        [pltpu.VMEM((CHUNK,), jnp.int32) for _ in range(QB)],
        [pltpu.VMEM((CHUNK, D), jnp.int32) for _ in range(DB)],
        [pltpu.VMEM((CHUNK, D), jnp.float32) for _ in range(DB)],
        pltpu.VMEM_SHARED((N // 2, D), jnp.int32),
        pltpu.VMEM_SHARED((N, D), jnp.float32),
        [pltpu.SemaphoreType.DMA for _ in range(QB)],
        [pltpu.SemaphoreType.DMA for _ in range(DB)],
        [pltpu.SemaphoreType.DMA for _ in range(DB)],
    ],
)
def _sc_aggregate(xb_hbm, src_hbm, dst_hbm, att_hbm, out_hbm,
                  srcs, dsts, atts, rids, bins, fouts, x_sh, ego_sh,
                  isems, gsems, ssems):
    c = lax.axis_index("c")
    s = lax.axis_index("s")
    wid = s * NC + c

    pltpu.sync_copy(xb_hbm.at[pl.ds(s * HSTRIPE, HSTRIPE)],
                    x_sh.at[pl.ds(s * HSTRIPE, HSTRIPE)])

    @pl.when(s == 0)
    def _():
        pltpu.sync_copy(xb_hbm.at[pl.ds(NS * HSTRIPE, N // 2 - NS * HSTRIPE)],
                        x_sh.at[pl.ds(NS * HSTRIPE, N // 2 - NS * HSTRIPE)])

    zvec = jnp.zeros((16,), jnp.float32)

    def _zero_row(i, _):
        fouts[0][i // 8, pl.ds((i % 8) * 16, 16)] = zvec
        return 0

    lax.fori_loop(0, CHUNK * 8, _zero_row, 0)
    stripe = s * ROWS_PER_SUB
    nfull = ROWS_PER_SUB // CHUNK
    for k in range(nfull):
        pltpu.sync_copy(fouts[0],
                        ego_sh.at[pl.ds(stripe + k * CHUNK, CHUNK)])
    tail = ROWS_PER_SUB - nfull * CHUNK
    if tail:
        pltpu.sync_copy(fouts[0].at[pl.ds(0, tail)],
                        ego_sh.at[pl.ds(stripe + nfull * CHUNK, tail)])
    plsc.subcore_barrier()

    def _prep_rows(q):
        rids[q][pl.ds(0, 16)] = lax.shift_right_logical(
            srcs[q][pl.ds(0, 16)], 1)

    def _scale(q, bin_, fout):
        a16 = atts[q][pl.ds(0, 16)]
        s16 = srcs[q][pl.ds(0, 16)]
        zero16 = jnp.zeros((16,), jnp.int32)
        for t in range(CHUNK):
            tvec = jnp.full((16,), t, jnp.int32)
            spl = jnp.take_along_axis(a16, tvec, axis=0)
            par = jnp.take_along_axis(s16, tvec, axis=0) & 1
            even = par == zero16
            for m in range(D // 32):
                vlo = bin_[t, pl.ds(m * 16, 16)]
                vhi = bin_[t, pl.ds(D // 2 + m * 16, 16)]
                v = jnp.where(even, vlo, vhi)
                vb = plsc.bitcast(v, jnp.bfloat16)
                lo, hi = plsc.unpack(vb, format=plsc.PackFormat.INTERLEAVED)
                fout[t, pl.ds(m * 32, 16)] = lo * spl
                fout[t, pl.ds(m * 32 + 16, 16)] = hi * spl

    for k in range(2):
        pltpu.async_copy(src_hbm.at[wid, k], srcs[k], isems[k])
        pltpu.async_copy(dst_hbm.at[wid, k], dsts[k], isems[k])
        pltpu.async_copy(att_hbm.at[wid, k], atts[k], isems[k])
    pltpu.make_async_copy(src_hbm.at[wid, 0], srcs[0], isems[0]).wait()
    pltpu.make_async_copy(dst_hbm.at[wid, 0], dsts[0], isems[0]).wait()
    pltpu.make_async_copy(att_hbm.at[wid, 0], atts[0], isems[0]).wait()
    _prep_rows(0)
    pltpu.async_copy(x_sh.at[rids[0]], bins[0], gsems[0])

    def _step(i4, _):
        for b in range(QB):
            i = i4 * QB + b
            d = b % DB
            od = (b + 1) % DB
            q = b
            nq = (b + 1) % QB
            pq = (b + QB - 1) % QB
            fq = (b + 2) % QB
            @pl.when(i >= 1)
            def _():
                pltpu.make_async_copy(
                    fouts[od], ego_sh.at[dsts[pq]], ssems[od]).wait()

            @pl.when(i + 2 < CPW)
            def _():
                pltpu.async_copy(src_hbm.at[wid, i + 2], srcs[fq],
                                 isems[fq])
                pltpu.async_copy(dst_hbm.at[wid, i + 2], dsts[fq],
                                 isems[fq])
                pltpu.async_copy(att_hbm.at[wid, i + 2], atts[fq],
                                 isems[fq])

            pltpu.make_async_copy(x_sh.at[rids[q]], bins[d],
                                  gsems[d]).wait()

            @pl.when(i + 1 < CPW)
            def _():
                pltpu.make_async_copy(
                    src_hbm.at[wid, i + 1], srcs[nq], isems[nq]).wait()
                pltpu.make_async_copy(
                    dst_hbm.at[wid, i + 1], dsts[nq], isems[nq]).wait()
                pltpu.make_async_copy(
                    att_hbm.at[wid, i + 1], atts[nq], isems[nq]).wait()
                _prep_rows(nq)
                pltpu.async_copy(x_sh.at[rids[nq]], bins[od], gsems[od])

            _scale(q, bins[d], fouts[d])
            pltpu.async_copy(fouts[d], ego_sh.at[dsts[q]], ssems[d],
                             add=True)
        return 0

    lax.fori_loop(0, CPW // QB, _step, 0)
    pltpu.make_async_copy(fouts[(CPW - 1) % DB],
                          ego_sh.at[dsts[(CPW - 1) % QB]],
                          ssems[(CPW - 1) % DB]).wait()
    plsc.subcore_barrier()

    pltpu.sync_copy(ego_sh.at[pl.ds(stripe, ROWS_PER_SUB)], out_hbm.at[c, s])


BLK = 1000


def _mlp_body(x_ref, p0_ref, p1_ref, w1_ref, b1_ref, w2_ref, b2_ref, o_ref):
    ego = p0_ref[...] + p1_ref[...]
    xv = x_ref[...]
    h1 = jnp.dot(xv + ego, w1_ref[...], preferred_element_type=jnp.float32) + b1_ref[...]
    h2 = jnp.dot(xv * ego, w2_ref[...], preferred_element_type=jnp.float32) + b2_ref[...]
    o_ref[...] = (jnp.where(h1 >= 0, h1, 0.01 * h1)
                  + jnp.where(h2 >= 0, h2, 0.01 * h2))


def _mlp(x, partials, W1, b1, W2, b2):
    grid = N // BLK
    return pl.pallas_call(
        _mlp_body,
        grid=(grid,),
        in_specs=[
            pl.BlockSpec((BLK, D), lambda i: (i, 0)),
            pl.BlockSpec((BLK, D), lambda i: (i, 0)),
            pl.BlockSpec((BLK, D), lambda i: (i + N // BLK, 0)),
            pl.BlockSpec((D, D), lambda i: (0, 0)),
            pl.BlockSpec((1, D), lambda i: (0, 0)),
            pl.BlockSpec((D, D), lambda i: (0, 0)),
            pl.BlockSpec((1, D), lambda i: (0, 0)),
        ],
        out_specs=pl.BlockSpec((BLK, D), lambda i: (i, 0)),
        out_shape=jax.ShapeDtypeStruct((N, D), jnp.float32),
    )(x, partials, partials, W1, b1, W2, b2)


def kernel(x, edge_index, attention, W1, b1, W2, b2):
    src = edge_index[0].astype(jnp.int32)
    dst = edge_index[1].astype(jnp.int32)
    pad = E_PAD - E
    src = jnp.concatenate([src, jnp.zeros((pad,), jnp.int32)])
    dst = jnp.concatenate([dst, jnp.zeros((pad,), jnp.int32)])
    att = jnp.concatenate([attention, jnp.zeros((pad,), jnp.float32)])
    srcr = src.reshape(NW, CPW, CHUNK)
    dstr = dst.reshape(NW, CPW, CHUNK)
    attr = att.reshape(NW, CPW, CHUNK)
    xb = x[:, _PERM].astype(jnp.bfloat16)
    xb = lax.bitcast_convert_type(xb.reshape(N, D // 2, 2), jnp.int32)
    xb = xb.reshape(N // 2, D)
    partials = _sc_aggregate(xb, srcr, dstr, attr).reshape(NC * N, D)
    return _mlp(x, partials, W1, b1.reshape(1, D), W2, b2.reshape(1, D))
